# baseline (speedup 1.0000x reference)
import functools
import jax
import jax.numpy as jnp
from jax import lax
from jax.experimental import pallas as pl
from jax.experimental.pallas import tpu as pltpu
from jax.experimental.pallas import tpu_sc as plsc

_B = 512
_N = 128
_F = 128
_H = 256
_MO = 64
_P = 64
_PN = _P * _N

_NC = 2
_NS = 16
_NW = _NC * _NS
_PPW = _B // _NW


_TRASH = _B * _N
_CNT1D = _B * _N + 16


def _sc_count_kernel(g_hbm, ones_hbm, zeros_hbm, out_hbm,
                     gv, onesv, zerov, shared):
    wid = lax.axis_index("c") * _NS + lax.axis_index("s")
    base = wid * _PPW * _N
    nloc = _PPW * _N
    pltpu.sync_copy(g_hbm.at[pl.ds(base, nloc)], gv)
    pltpu.sync_copy(ones_hbm, onesv)
    pltpu.sync_copy(zeros_hbm, zerov)
    pltpu.sync_copy(zeros_hbm, shared.at[pl.ds(base, nloc)])
    pltpu.sync_copy(onesv, shared.at[gv], add=True)
    pltpu.sync_copy(zerov, shared.at[gv], add=True)
    pltpu.sync_copy(shared.at[pl.ds(base, nloc)], out_hbm.at[pl.ds(base, nloc)])


def _sc_count(g_flat):
    mesh = plsc.VectorSubcoreMesh(core_axis_name="c", subcore_axis_name="s")
    nloc = _PPW * _N
    return pl.kernel(
        _sc_count_kernel,
        mesh=mesh,
        out_type=jax.ShapeDtypeStruct((_B * _N,), jnp.float32),
        scratch_types=[
            pltpu.VMEM((nloc,), jnp.int32),
            pltpu.VMEM((nloc,), jnp.float32),
            pltpu.VMEM((nloc,), jnp.float32),
            pltpu.VMEM_SHARED((_CNT1D,), jnp.float32),
        ],
    )(g_flat, jnp.ones((nloc,), jnp.float32), jnp.zeros((nloc,), jnp.float32))


def _fused_kernel(trees_ref, count_ref, rootidx_ref, Wemb_ref, bemb_ref,
                  Wcat_ref, bmlp_ref, Wck_ref, bck_ref,
                  cost_ref, card_ref, mlp_ref):
    f32 = jnp.float32
    bf16 = jnp.bfloat16
    i32 = jnp.int32

    Wemb = Wemb_ref[...].astype(bf16)
    A = jnp.transpose(trees_ref[...].astype(bf16), (0, 2, 1)).reshape(_PN, _F)
    proj = jax.lax.dot_general(A, Wemb, (((1,), (0,)), ((), ())),
                               preferred_element_type=f32)
    E = jnp.maximum(proj + bemb_ref[...], 0.0).astype(bf16)

    EW = jax.lax.dot_general(E, Wcat_ref[...], (((1,), (0,)), ((), ())),
                             preferred_element_type=f32)
    S_T = EW[:, 0:1].reshape(_P, _N)

    countT = count_ref[0]

    mx = jnp.max(jnp.where(countT > 0.0, S_T, f32(-1e30)), axis=1,
                 keepdims=True)
    e = countT * jnp.exp(jnp.minimum(S_T - mx, 0.0))
    W_T = e / (jnp.sum(e, axis=1, keepdims=True) + f32(1e-9))

    lane_g = jax.lax.broadcasted_iota(i32, (_P, _PN), 1)
    row_p = jax.lax.broadcasted_iota(i32, (_P, _PN), 0)
    planmask = jax.lax.shift_right_logical(lane_g, 7) == row_p
    W_blk = jnp.where(planmask, W_T.reshape(1, _PN), 0.0)
    target = row_p * _N + rootidx_ref[0]
    R_sel = (lane_g == target).astype(f32)
    Sel = jnp.concatenate([W_blk, R_sel], axis=0).astype(bf16)
    RP = jax.lax.dot_general(Sel, EW[:, 1:129].astype(bf16),
                             (((1,), (0,)), ((), ())),
                             preferred_element_type=f32)

    mlp = jnp.maximum(RP[:_P, 0:_MO] + RP[_P:, _MO:2 * _MO]
                      + bmlp_ref[...], 0.0)
    mlp_ref[...] = mlp
    hk = jax.lax.dot_general(mlp, Wck_ref[...], (((1,), (0,)), ((), ())),
                             preferred_element_type=f32) + bck_ref[...]
    cost_ref[...] = hk[:, 0:1]
    card_ref[...] = hk[:, 1:2]


def kernel(trees, indexes, mask_padding, W_emb, b_emb, w_attn, W_mlp, b_mlp,
           W_cost, b_cost, W_card, b_card):
    f32 = jnp.float32
    idx = indexes.astype(jnp.int32)
    gbins = jnp.where(
        mask_padding, jnp.int32(_TRASH),
        idx + jnp.arange(_B, dtype=jnp.int32)[:, None] * _N)
    count = _sc_count(gbins.reshape(_B * _N)).reshape(_B // _P, _P, _N)
    rootidx = idx[:, 1].reshape(_B // _P, _P, 1)
    bemb2 = b_emb.reshape(1, _H).astype(f32)
    Wcat = jnp.concatenate(
        [w_attn.reshape(_H, 1), W_mlp[_H:], W_mlp[:_H]],
        axis=1).astype(jnp.bfloat16)
    bmlp2 = b_mlp.reshape(1, _MO).astype(f32)
    Wck = jnp.concatenate([W_cost, W_card], axis=1)
    bck = jnp.concatenate([b_cost, b_card]).reshape(1, 2).astype(f32)

    grid = (_B // _P,)

    out = pl.pallas_call(
        _fused_kernel,
        grid=grid,
        in_specs=[
            pl.BlockSpec((_P, _F, _N), lambda i: (i, 0, 0)),
            pl.BlockSpec((1, _P, _N), lambda i: (i, 0, 0)),
            pl.BlockSpec((1, _P, 1), lambda i: (i, 0, 0)),
            pl.BlockSpec((_F, _H), lambda i: (0, 0)),
            pl.BlockSpec((1, _H), lambda i: (0, 0)),
            pl.BlockSpec((_H, 129), lambda i: (0, 0)),
            pl.BlockSpec((1, _MO), lambda i: (0, 0)),
            pl.BlockSpec((_MO, 2), lambda i: (0, 0)),
            pl.BlockSpec((1, 2), lambda i: (0, 0)),
        ],
        out_specs=[
            pl.BlockSpec((_P, 1), lambda i: (i, 0)),
            pl.BlockSpec((_P, 1), lambda i: (i, 0)),
            pl.BlockSpec((_P, _MO), lambda i: (i, 0)),
        ],
        out_shape=[
            jax.ShapeDtypeStruct((_B, 1), f32),
            jax.ShapeDtypeStruct((_B, 1), f32),
            jax.ShapeDtypeStruct((_B, _MO), f32),
        ],
        compiler_params=pltpu.CompilerParams(
            dimension_semantics=("arbitrary",)),
    )(trees, count, rootidx, W_emb, bemb2, Wcat, bmlp2, Wck, bck)
    pred_cost, pred_card, mlp_out = out
    return pred_cost, pred_card, mlp_out


# speedup vs baseline: 1.0008x; 1.0008x over previous
import functools
import jax
import jax.numpy as jnp
from jax import lax
from jax.experimental import pallas as pl
from jax.experimental.pallas import tpu as pltpu
from jax.experimental.pallas import tpu_sc as plsc

_B = 512
_N = 128
_F = 128
_H = 256
_MO = 64
_P = 64
_PN = _P * _N

_NC = 2
_NS = 16
_NW = _NC * _NS
_PPW = _B // _NW


_TRASH = _B * _N
_CNT1D = _B * _N + 16


def _sc_count_kernel(g_hbm, ones_hbm, zeros_hbm, out_hbm,
                     gv, onesv, zerov, shared, sem1, sem2, sem3, sem4):
    wid = lax.axis_index("c") * _NS + lax.axis_index("s")
    base = wid * _PPW * _N
    nloc = _PPW * _N
    c1 = pltpu.async_copy(g_hbm.at[pl.ds(base, nloc)], gv, sem1)
    c2 = pltpu.async_copy(ones_hbm, onesv, sem2)
    c3 = pltpu.async_copy(zeros_hbm, zerov, sem3)
    c4 = pltpu.async_copy(zeros_hbm, shared.at[pl.ds(base, nloc)], sem4)
    c1.wait(); c2.wait(); c3.wait(); c4.wait()
    pltpu.sync_copy(onesv, shared.at[gv], add=True)
    pltpu.sync_copy(zerov, shared.at[gv], add=True)
    pltpu.sync_copy(shared.at[pl.ds(base, nloc)], out_hbm.at[pl.ds(base, nloc)])


def _sc_count(g_flat):
    mesh = plsc.VectorSubcoreMesh(core_axis_name="c", subcore_axis_name="s")
    nloc = _PPW * _N
    return pl.kernel(
        _sc_count_kernel,
        mesh=mesh,
        out_type=jax.ShapeDtypeStruct((_B * _N,), jnp.float32),
        scratch_types=[
            pltpu.VMEM((nloc,), jnp.int32),
            pltpu.VMEM((nloc,), jnp.float32),
            pltpu.VMEM((nloc,), jnp.float32),
            pltpu.VMEM_SHARED((_CNT1D,), jnp.float32),
            pltpu.SemaphoreType.DMA,
            pltpu.SemaphoreType.DMA,
            pltpu.SemaphoreType.DMA,
            pltpu.SemaphoreType.DMA,
        ],
    )(g_flat, jnp.ones((nloc,), jnp.float32), jnp.zeros((nloc,), jnp.float32))


def _fused_kernel(trees_ref, count_ref, rootidx_ref, Wemb_ref, bemb_ref,
                  Wcat_ref, bmlp_ref, Wck_ref, bck_ref,
                  cost_ref, card_ref, mlp_ref):
    f32 = jnp.float32
    bf16 = jnp.bfloat16
    i32 = jnp.int32

    Wemb = Wemb_ref[...].astype(bf16)
    A = jnp.transpose(trees_ref[...].astype(bf16), (0, 2, 1)).reshape(_PN, _F)
    proj = jax.lax.dot_general(A, Wemb, (((1,), (0,)), ((), ())),
                               preferred_element_type=f32)
    E = jnp.maximum(proj + bemb_ref[...], 0.0).astype(bf16)

    EW = jax.lax.dot_general(E, Wcat_ref[...], (((1,), (0,)), ((), ())),
                             preferred_element_type=f32)
    S_T = EW[:, 0:1].reshape(_P, _N)

    countT = count_ref[0]

    mx = jnp.max(jnp.where(countT > 0.0, S_T, f32(-1e30)), axis=1,
                 keepdims=True)
    e = countT * jnp.exp(jnp.minimum(S_T - mx, 0.0))
    W_T = e / (jnp.sum(e, axis=1, keepdims=True) + f32(1e-9))

    lane_g = jax.lax.broadcasted_iota(i32, (_P, _PN), 1)
    row_p = jax.lax.broadcasted_iota(i32, (_P, _PN), 0)
    planmask = jax.lax.shift_right_logical(lane_g, 7) == row_p
    W_blk = jnp.where(planmask, W_T.reshape(1, _PN), 0.0)
    target = row_p * _N + rootidx_ref[0]
    R_sel = (lane_g == target).astype(f32)
    Sel = jnp.concatenate([W_blk, R_sel], axis=0).astype(bf16)
    RP = jax.lax.dot_general(Sel, EW[:, 1:129].astype(bf16),
                             (((1,), (0,)), ((), ())),
                             preferred_element_type=f32)

    mlp = jnp.maximum(RP[:_P, 0:_MO] + RP[_P:, _MO:2 * _MO]
                      + bmlp_ref[...], 0.0)
    mlp_ref[...] = mlp
    hk = jax.lax.dot_general(mlp, Wck_ref[...], (((1,), (0,)), ((), ())),
                             preferred_element_type=f32) + bck_ref[...]
    cost_ref[...] = hk[:, 0:1]
    card_ref[...] = hk[:, 1:2]


def kernel(trees, indexes, mask_padding, W_emb, b_emb, w_attn, W_mlp, b_mlp,
           W_cost, b_cost, W_card, b_card):
    f32 = jnp.float32
    idx = indexes.astype(jnp.int32)
    gbins = jnp.where(
        mask_padding, jnp.int32(_TRASH),
        idx + jnp.arange(_B, dtype=jnp.int32)[:, None] * _N)
    count = _sc_count(gbins.reshape(_B * _N)).reshape(_B // _P, _P, _N)
    rootidx = idx[:, 1].reshape(_B // _P, _P, 1)
    bemb2 = b_emb.reshape(1, _H).astype(f32)
    Wcat = jnp.concatenate(
        [w_attn.reshape(_H, 1), W_mlp[_H:], W_mlp[:_H]],
        axis=1).astype(jnp.bfloat16)
    bmlp2 = b_mlp.reshape(1, _MO).astype(f32)
    Wck = jnp.concatenate([W_cost, W_card], axis=1)
    bck = jnp.concatenate([b_cost, b_card]).reshape(1, 2).astype(f32)

    grid = (_B // _P,)

    out = pl.pallas_call(
        _fused_kernel,
        grid=grid,
        in_specs=[
            pl.BlockSpec((_P, _F, _N), lambda i: (i, 0, 0)),
            pl.BlockSpec((1, _P, _N), lambda i: (i, 0, 0)),
            pl.BlockSpec((1, _P, 1), lambda i: (i, 0, 0)),
            pl.BlockSpec((_F, _H), lambda i: (0, 0)),
            pl.BlockSpec((1, _H), lambda i: (0, 0)),
            pl.BlockSpec((_H, 129), lambda i: (0, 0)),
            pl.BlockSpec((1, _MO), lambda i: (0, 0)),
            pl.BlockSpec((_MO, 2), lambda i: (0, 0)),
            pl.BlockSpec((1, 2), lambda i: (0, 0)),
        ],
        out_specs=[
            pl.BlockSpec((_P, 1), lambda i: (i, 0)),
            pl.BlockSpec((_P, 1), lambda i: (i, 0)),
            pl.BlockSpec((_P, _MO), lambda i: (i, 0)),
        ],
        out_shape=[
            jax.ShapeDtypeStruct((_B, 1), f32),
            jax.ShapeDtypeStruct((_B, 1), f32),
            jax.ShapeDtypeStruct((_B, _MO), f32),
        ],
        compiler_params=pltpu.CompilerParams(
            dimension_semantics=("arbitrary",)),
    )(trees, count, rootidx, W_emb, bemb2, Wcat, bmlp2, Wck, bck)
    pred_cost, pred_card, mlp_out = out
    return pred_cost, pred_card, mlp_out


# speedup vs baseline: 1.6509x; 1.6497x over previous
import jax
import jax.numpy as jnp
from jax.experimental import pallas as pl
from jax.experimental.pallas import tpu as pltpu

_B = 512
_N = 128
_F = 128
_H = 256
_MO = 64
_P = 64
_PN = _P * _N


def _fused_kernel(trees_ref, idxm_ref, rootidx_ref, Wemb_ref, bemb_ref,
                  Wcat_ref, bmlp_ref, Wck_ref, bck_ref,
                  cost_ref, card_ref, mlp_ref):
    f32 = jnp.float32
    bf16 = jnp.bfloat16
    i32 = jnp.int32

    Wemb = Wemb_ref[...].astype(bf16)
    A = jnp.transpose(trees_ref[...].astype(bf16), (0, 2, 1)).reshape(_PN, _F)
    proj = jax.lax.dot_general(A, Wemb, (((1,), (0,)), ((), ())),
                               preferred_element_type=f32)
    E = jnp.maximum(proj + bemb_ref[...], 0.0).astype(bf16)

    EW = jax.lax.dot_general(E, Wcat_ref[...], (((1,), (0,)), ((), ())),
                             preferred_element_type=f32)
    S_T = EW[:, 0:1].reshape(_P, _N)

    idxr = idxm_ref[0]
    iota_m = jax.lax.broadcasted_iota(i32, (_N, _PN), 0)
    OTV = (iota_m == idxr).astype(bf16)
    blk = (jax.lax.shift_right_logical(
        jax.lax.broadcasted_iota(i32, (_PN, _P), 0), 7)
        == jax.lax.broadcasted_iota(i32, (_PN, _P), 1)).astype(bf16)
    count = jax.lax.dot_general(OTV, blk, (((1,), (0,)), ((), ())),
                                preferred_element_type=f32)
    countT = jnp.transpose(count)

    mx = jnp.max(jnp.where(countT > 0.0, S_T, f32(-1e30)), axis=1,
                 keepdims=True)
    e = countT * jnp.exp(jnp.minimum(S_T - mx, 0.0))
    W_T = e / (jnp.sum(e, axis=1, keepdims=True) + f32(1e-9))

    lane_g = jax.lax.broadcasted_iota(i32, (_P, _PN), 1)
    row_p = jax.lax.broadcasted_iota(i32, (_P, _PN), 0)
    planmask = jax.lax.shift_right_logical(lane_g, 7) == row_p
    W_blk = jnp.where(planmask, W_T.reshape(1, _PN), 0.0)
    target = row_p * _N + rootidx_ref[0]
    R_sel = (lane_g == target).astype(f32)
    Sel = jnp.concatenate([W_blk, R_sel], axis=0).astype(bf16)
    RP = jax.lax.dot_general(Sel, EW[:, 1:129].astype(bf16),
                             (((1,), (0,)), ((), ())),
                             preferred_element_type=f32)

    mlp = jnp.maximum(RP[:_P, 0:_MO] + RP[_P:, _MO:2 * _MO]
                      + bmlp_ref[...], 0.0)
    mlp_ref[...] = mlp
    hk = jax.lax.dot_general(mlp, Wck_ref[...], (((1,), (0,)), ((), ())),
                             preferred_element_type=f32) + bck_ref[...]
    cost_ref[...] = hk[:, 0:1]
    card_ref[...] = hk[:, 1:2]


def kernel(trees, indexes, mask_padding, W_emb, b_emb, w_attn, W_mlp, b_mlp,
           W_cost, b_cost, W_card, b_card):
    f32 = jnp.float32
    idx = indexes.astype(jnp.int32)
    idx_masked = jnp.where(mask_padding, jnp.int32(-1), idx)
    idxm_flat = idx_masked.reshape(_B // _P, 1, _PN)
    rootidx = idx[:, 1].reshape(_B // _P, _P, 1)
    bemb2 = b_emb.reshape(1, _H).astype(f32)
    Wcat = jnp.concatenate(
        [w_attn.reshape(_H, 1), W_mlp[_H:], W_mlp[:_H]],
        axis=1).astype(jnp.bfloat16)
    bmlp2 = b_mlp.reshape(1, _MO).astype(f32)
    Wck = jnp.concatenate([W_cost, W_card], axis=1)
    bck = jnp.concatenate([b_cost, b_card]).reshape(1, 2).astype(f32)

    grid = (_B // _P,)

    out = pl.pallas_call(
        _fused_kernel,
        grid=grid,
        in_specs=[
            pl.BlockSpec((_P, _F, _N), lambda i: (i, 0, 0)),
            pl.BlockSpec((1, 1, _PN), lambda i: (i, 0, 0)),
            pl.BlockSpec((1, _P, 1), lambda i: (i, 0, 0)),
            pl.BlockSpec((_F, _H), lambda i: (0, 0)),
            pl.BlockSpec((1, _H), lambda i: (0, 0)),
            pl.BlockSpec((_H, 129), lambda i: (0, 0)),
            pl.BlockSpec((1, _MO), lambda i: (0, 0)),
            pl.BlockSpec((_MO, 2), lambda i: (0, 0)),
            pl.BlockSpec((1, 2), lambda i: (0, 0)),
        ],
        out_specs=[
            pl.BlockSpec((_P, 1), lambda i: (i, 0)),
            pl.BlockSpec((_P, 1), lambda i: (i, 0)),
            pl.BlockSpec((_P, _MO), lambda i: (i, 0)),
        ],
        out_shape=[
            jax.ShapeDtypeStruct((_B, 1), f32),
            jax.ShapeDtypeStruct((_B, 1), f32),
            jax.ShapeDtypeStruct((_B, _MO), f32),
        ],
        compiler_params=pltpu.CompilerParams(
            dimension_semantics=("arbitrary",)),
    )(trees, idxm_flat, rootidx, W_emb, bemb2, Wcat, bmlp2, Wck, bck)
    pred_cost, pred_card, mlp_out = out
    return pred_cost, pred_card, mlp_out
